# baseline (speedup 1.0000x reference)
import functools

import jax
import jax.numpy as jnp
from jax import lax
from jax.experimental import pallas as pl
from jax.experimental.pallas import tpu as pltpu
from jax.experimental.pallas import tpu_sc as plsc

_TAU = 10.0
_DT = 1.0
_THRESH = 0.5
_VMAX = 1.0
_ATOL = 1e-5
_RTOL = 1e-8

_EB_A = 6144
_EB_C = 12288
_PB = 128

_NC = 2
_NS = 16
_NW = _NC * _NS
_IR = 128


def _mesh():
    return plsc.VectorSubcoreMesh(
        core_axis_name="c", subcore_axis_name="s",
        num_cores=_NC, num_subcores=_NS)


def _sc_params():
    return pltpu.CompilerParams(needs_layout_passes=False,
                                use_tc_tiling_on_sc=False)


def _spikes_kernel(st_lo, sv_lo, l_lo, w_lo, st_hi, sv_hi, l_hi, w_hi,
                   spkt_ref):
    def spk(st_ref, sv_ref, l_ref, w_ref):
        st = st_ref[...]
        lvals = l_ref[0]
        arrived = jnp.abs(st - lvals) <= (_ATOL + _RTOL * jnp.abs(lvals))
        return jnp.where(arrived, sv_ref[...] * w_ref[0], 0.0)

    lo = spk(st_lo, sv_lo, l_lo, w_lo)
    hi = spk(st_hi, sv_hi, l_hi, w_hi)
    spkt_ref[...] = jnp.concatenate([lo.T, hi.T], axis=1)


def _make_sc_scatter(b, e, n, ce):
    eper = e // _NW
    nchunks = eper // ce
    assert nchunks % 2 == 0
    kr = ce // _IR
    zrows = n // _NS
    zc = 8
    assert zrows % zc == 0

    @functools.partial(
        pl.kernel, mesh=_mesh(),
        out_type=jax.ShapeDtypeStruct((_NC, n, b), jnp.float32),
        scratch_types=[
            pltpu.VMEM((2, kr, _IR, b), jnp.float32),
            pltpu.VMEM((2, kr, _IR), jnp.int32),
            pltpu.VMEM((zc, b), jnp.float32),
            pltpu.VMEM_SHARED((n, b), jnp.float32),
            pltpu.SemaphoreType.DMA,
            pltpu.SemaphoreType.DMA,
            pltpu.SemaphoreType.DMA,
            pltpu.SemaphoreType.DMA,
        ],
        compiler_params=_sc_params(),
    )
    def k(spkt_hbm, tgt3_hbm, isynt_hbm, val_v, idx_v, zbuf, shared,
          in0, in1, sc0, sc1):
        cid = lax.axis_index("c")
        sid = lax.axis_index("s")
        wid = sid * _NC + cid
        rb0 = (wid * eper) // _IR
        insems = (in0, in1)
        scsems = (sc0, sc1)

        for r in range(zc):
            for q in range(b // 16):
                zbuf[r, pl.ds(q * 16, 16)] = jnp.zeros((16,), jnp.float32)
        for z in range(zrows // zc):
            pltpu.sync_copy(
                zbuf, shared.at[pl.ds(sid * zrows + z * zc, zc)])
        plsc.subcore_barrier()

        def start_in(ci, p):
            pltpu.async_copy(spkt_hbm.at[pl.ds(rb0 + ci * kr, kr)],
                             val_v.at[p], insems[p])
            pltpu.async_copy(tgt3_hbm.at[pl.ds(rb0 + ci * kr, kr)],
                             idx_v.at[p], insems[p])

        def wait_in(p):
            pltpu.make_async_copy(spkt_hbm.at[pl.ds(0, kr)], val_v.at[p],
                                  insems[p]).wait()
            pltpu.make_async_copy(tgt3_hbm.at[pl.ds(0, kr)], idx_v.at[p],
                                  insems[p]).wait()

        def issue_scatter(p):
            for j in range(kr):
                pltpu.async_copy(val_v.at[p, j],
                                 shared.at[idx_v.at[p, j]], scsems[p],
                                 add=True)

        def drain_scatter(p):
            for j in range(kr):
                pltpu.make_async_copy(val_v.at[p, j],
                                      shared.at[idx_v.at[p, j]],
                                      scsems[p]).wait()

        start_in(0, 0)

        def pair_body(i, _):
            start_in(2 * i + 1, 1)
            wait_in(0)
            issue_scatter(0)
            drain_scatter(0)

            @pl.when(2 * i + 2 < nchunks)
            def _():
                start_in(2 * i + 2, 0)
            wait_in(1)
            issue_scatter(1)
            drain_scatter(1)
            return 0
        lax.fori_loop(0, nchunks // 2, pair_body, 0)

        plsc.subcore_barrier()

        @pl.when(sid == 0)
        def _():
            pltpu.sync_copy(shared, isynt_hbm.at[cid])

    return k


def _neuron_kernel(nh, no, tail_pad, isynt_ref, vm_ref, acc_ref, inp_ref,
                   phase_ref, vexct_ref, tail_ref):
    inject = (phase_ref[...] == 2).astype(jnp.float32)
    inp = inp_ref[...]
    b = inp.shape[0]
    i_syn = (isynt_ref[0] + isynt_ref[1]).T
    i_inj = jnp.concatenate(
        [inp * inject, jnp.zeros((b, no), jnp.float32)], axis=1)
    i_syn = i_syn + i_inj
    vm = vm_ref[...]
    vm1 = vm + (i_syn - vm) * (_DT / _TAU)
    v_exc = jnp.maximum(0.0, vm1 - _THRESH)
    fired = (v_exc > 0).astype(jnp.float32)
    vm2 = vm1 - vm1 * fired + 0.2 * fired
    acc1 = acc_ref[...] + vm1[:, -no:]
    spike_rate = jnp.mean(fired, axis=1, keepdims=True)
    input_norm = jnp.sqrt(jnp.sum(inp * inp, axis=1, keepdims=True)) * inject
    vexct_ref[...] = v_exc.T
    tail_ref[...] = jnp.concatenate(
        [vm2, acc1, inject, spike_rate, input_norm,
         jnp.zeros((b, tail_pad), jnp.float32)], axis=1)


def _make_sc_gather(b, e, n, ce):
    eper = e // _NW
    nchunks = eper // ce
    assert nchunks % 2 == 0
    kr = ce // _IR
    srows = n // _NS

    @functools.partial(
        pl.kernel, mesh=_mesh(),
        out_type=jax.ShapeDtypeStruct((e // _IR, _IR, b), jnp.float32),
        scratch_types=[
            pltpu.VMEM((2, kr, _IR, b), jnp.float32),
            pltpu.VMEM((2, kr, _IR), jnp.int32),
            pltpu.VMEM_SHARED((n, b), jnp.float32),
            pltpu.SemaphoreType.DMA,
            pltpu.SemaphoreType.DMA,
            pltpu.SemaphoreType.DMA,
            pltpu.SemaphoreType.DMA,
            pltpu.SemaphoreType.DMA,
            pltpu.SemaphoreType.DMA,
        ],
        compiler_params=_sc_params(),
    )
    def k(vexct_hbm, src3_hbm, gt_hbm, g_v, idx_v, shared,
          in0, in1, ga0, ga1, out0, out1):
        cid = lax.axis_index("c")
        sid = lax.axis_index("s")
        wid = sid * _NC + cid
        rb0 = (wid * eper) // _IR
        insems = (in0, in1)
        gasems = (ga0, ga1)
        outsems = (out0, out1)

        pltpu.sync_copy(vexct_hbm.at[pl.ds(sid * srows, srows)],
                        shared.at[pl.ds(sid * srows, srows)])
        plsc.subcore_barrier()

        def start_in(ci, p):
            pltpu.async_copy(src3_hbm.at[pl.ds(rb0 + ci * kr, kr)],
                             idx_v.at[p], insems[p])

        def wait_in(p):
            pltpu.make_async_copy(src3_hbm.at[pl.ds(0, kr)], idx_v.at[p],
                                  insems[p]).wait()

        def issue_gather(p):
            for j in range(kr):
                pltpu.async_copy(shared.at[idx_v.at[p, j]],
                                 g_v.at[p, j], gasems[p])

        def drain_gather(p):
            for j in range(kr):
                pltpu.make_async_copy(shared.at[idx_v.at[p, j]],
                                      g_v.at[p, j], gasems[p]).wait()

        def start_out(ci, p):
            pltpu.async_copy(g_v.at[p],
                             gt_hbm.at[pl.ds(rb0 + ci * kr, kr)],
                             outsems[p])

        def drain_out(p):
            pltpu.make_async_copy(g_v.at[p], gt_hbm.at[pl.ds(0, kr)],
                                  outsems[p]).wait()

        start_in(0, 0)

        def pair_body(i, _):
            start_in(2 * i + 1, 1)
            wait_in(0)

            @pl.when(i > 0)
            def _():
                drain_out(0)
            issue_gather(0)
            drain_gather(0)
            start_out(2 * i, 0)

            @pl.when(2 * i + 2 < nchunks)
            def _():
                start_in(2 * i + 2, 0)
            wait_in(1)

            @pl.when(i > 0)
            def _():
                drain_out(1)
            issue_gather(1)
            drain_gather(1)
            start_out(2 * i + 1, 1)
            return 0
        lax.fori_loop(0, nchunks // 2, pair_body, 0)
        drain_out(0)
        drain_out(1)

    return k


def _pass2_kernel(n_st_blocks, st_ref, sv_ref, l_ref, gt_ref, tail_ref,
                  out_ref, sv_stash):
    i = pl.program_id(0)
    last = 2 * n_st_blocks

    def half(g):
        st = st_ref[...]
        sv = sv_ref[...]
        lvals = l_ref[0]
        arrived = jnp.abs(st - lvals) <= (_ATOL + _RTOL * jnp.abs(lvals))
        stz = jnp.where(arrived, 0.0, st)
        svz = jnp.where(arrived, 0.0, sv)
        new = (g > 0) & (st == 0)
        st2 = stz + jnp.where(stz > 0, _DT * _VMAX, 0.0) \
                  + jnp.where(new, _DT * _VMAX, 0.0)
        sv2 = svz + jnp.where(new, g, 0.0)
        out_ref[...] = st2
        sv_stash[...] = sv2

    b = st_ref.shape[0]

    @pl.when((i < last) & (i % 4 == 0))
    def _():
        half(gt_ref[...][:, :b].T)

    @pl.when((i < last) & (i % 4 == 2))
    def _():
        half(gt_ref[...][:, b:].T)

    @pl.when((i < last) & (i % 2 == 1))
    def _():
        out_ref[...] = sv_stash[...]

    @pl.when(i == last)
    def _():
        out_ref[...] = tail_ref[...]


def _pack_idx(idx, e):
    return idx.reshape(2, e // _IR, _IR // 2).transpose(1, 2, 0).reshape(
        e // _IR, _IR)


def kernel(syn_travel, syn_value, vm, acc, input_current, L_e, W_e, phase,
           src, tgt):
    b, e = syn_travel.shape
    n = vm.shape[1]
    nh = input_current.shape[1]
    no = acc.shape[1]
    f32 = jnp.float32
    half = e // 2

    eba = _EB_A
    nblk_a = half // eba
    l3 = L_e.reshape(2 * nblk_a, 1, eba)
    w3 = W_e.reshape(2 * nblk_a, 1, eba)

    def lo2(i):
        return (0, i)

    def hi2(i):
        return (0, i + nblk_a)

    def lo3(i):
        return (i, 0, 0)

    def hi3(i):
        return (i + nblk_a, 0, 0)

    spikes_t = pl.pallas_call(
        _spikes_kernel,
        grid=(nblk_a,),
        in_specs=[
            pl.BlockSpec((b, eba), lo2),
            pl.BlockSpec((b, eba), lo2),
            pl.BlockSpec((1, 1, eba), lo3),
            pl.BlockSpec((1, 1, eba), lo3),
            pl.BlockSpec((b, eba), hi2),
            pl.BlockSpec((b, eba), hi2),
            pl.BlockSpec((1, 1, eba), hi3),
            pl.BlockSpec((1, 1, eba), hi3),
        ],
        out_specs=pl.BlockSpec((eba, _PB), lambda i: (i, 0)),
        out_shape=jax.ShapeDtypeStruct((half, _PB), f32),
    )(syn_travel, syn_value, l3, w3, syn_travel, syn_value, l3, w3)

    ce = 384
    isynt = _make_sc_scatter(b, e, n, ce)(
        spikes_t.reshape(e // _IR, _IR, b), _pack_idx(tgt, e))

    tail_cols = n + no + 3
    tail_pad = _EB_C - tail_cols
    vexct, tail = pl.pallas_call(
        functools.partial(_neuron_kernel, nh, no, tail_pad),
        out_shape=[
            jax.ShapeDtypeStruct((n, b), f32),
            jax.ShapeDtypeStruct((b, _EB_C), f32),
        ],
    )(isynt, vm, acc, input_current, phase.reshape(b, 1))

    gathered_t = _make_sc_gather(b, e, n, ce)(
        vexct, _pack_idx(src, e)).reshape(half, _PB)

    ebc = _EB_C
    nblk_c = e // ebc
    nhalf_c = nblk_c // 2
    out_cols = 2 * e + tail_cols
    l3c = L_e.reshape(nblk_c, 1, ebc)

    def q_of(i):
        return jnp.minimum(i // 4, nhalf_c - 1)

    def st_map2(i):
        return (0, q_of(i) + nhalf_c * ((i % 4) // 2))

    def st_map3(i):
        return (q_of(i) + nhalf_c * ((i % 4) // 2), 0, 0)

    def gt_map(i):
        return (q_of(i), 0)

    def out_map(i):
        ph = i % 4
        blk = q_of(i) + nhalf_c * jnp.where(
            ph == 0, 0, jnp.where(ph == 1, 2, jnp.where(ph == 2, 1, 3)))
        return (0, jnp.where(i == 2 * nblk_c, 2 * nblk_c, blk))

    out = pl.pallas_call(
        functools.partial(_pass2_kernel, nblk_c),
        grid=(2 * nblk_c + 1,),
        in_specs=[
            pl.BlockSpec((b, ebc), st_map2),
            pl.BlockSpec((b, ebc), st_map2),
            pl.BlockSpec((1, 1, ebc), st_map3),
            pl.BlockSpec((ebc, _PB), gt_map),
            pl.BlockSpec((b, _EB_C), lambda i: (0, 0)),
        ],
        out_specs=pl.BlockSpec((b, ebc), out_map),
        out_shape=jax.ShapeDtypeStruct((b, out_cols), f32),
        scratch_shapes=[pltpu.VMEM((b, ebc), f32)],
    )(syn_travel, syn_value, l3c, gathered_t, tail)

    return out


# speedup vs baseline: 1.0023x; 1.0023x over previous
import functools

import jax
import jax.numpy as jnp
from jax import lax
from jax.experimental import pallas as pl
from jax.experimental.pallas import tpu as pltpu
from jax.experimental.pallas import tpu_sc as plsc

_TAU = 10.0
_DT = 1.0
_THRESH = 0.5
_VMAX = 1.0
_ATOL = 1e-5
_RTOL = 1e-8

_EB_A = 6144
_EB_C = 6144
_PB = 128

_NC = 2
_NS = 16
_NW = _NC * _NS
_IR = 128


def _mesh():
    return plsc.VectorSubcoreMesh(
        core_axis_name="c", subcore_axis_name="s",
        num_cores=_NC, num_subcores=_NS)


def _sc_params():
    return pltpu.CompilerParams(needs_layout_passes=False,
                                use_tc_tiling_on_sc=False)


def _spikes_kernel(st_lo, sv_lo, l_lo, w_lo, st_hi, sv_hi, l_hi, w_hi,
                   spkt_ref):
    def spk(st_ref, sv_ref, l_ref, w_ref):
        st = st_ref[...]
        lvals = l_ref[0]
        arrived = jnp.abs(st - lvals) <= (_ATOL + _RTOL * jnp.abs(lvals))
        return jnp.where(arrived, sv_ref[...] * w_ref[0], 0.0)

    lo = spk(st_lo, sv_lo, l_lo, w_lo)
    hi = spk(st_hi, sv_hi, l_hi, w_hi)
    spkt_ref[...] = jnp.concatenate([lo.T, hi.T], axis=1)


def _make_sc_scatter(b, e, n, ce):
    eper = e // _NW
    nchunks = eper // ce
    assert nchunks % 2 == 0
    kr = ce // _IR
    zrows = n // _NS
    zc = 8
    assert zrows % zc == 0

    @functools.partial(
        pl.kernel, mesh=_mesh(),
        out_type=jax.ShapeDtypeStruct((_NC, n, b), jnp.float32),
        scratch_types=[
            pltpu.VMEM((2, kr, _IR, b), jnp.float32),
            pltpu.VMEM((2, kr, _IR), jnp.int32),
            pltpu.VMEM((zc, b), jnp.float32),
            pltpu.VMEM_SHARED((n, b), jnp.float32),
            pltpu.SemaphoreType.DMA,
            pltpu.SemaphoreType.DMA,
            pltpu.SemaphoreType.DMA,
            pltpu.SemaphoreType.DMA,
        ],
        compiler_params=_sc_params(),
    )
    def k(spkt_hbm, tgt3_hbm, isynt_hbm, val_v, idx_v, zbuf, shared,
          in0, in1, sc0, sc1):
        cid = lax.axis_index("c")
        sid = lax.axis_index("s")
        wid = sid * _NC + cid
        rb0 = (wid * eper) // _IR
        insems = (in0, in1)
        scsems = (sc0, sc1)

        for r in range(zc):
            for q in range(b // 16):
                zbuf[r, pl.ds(q * 16, 16)] = jnp.zeros((16,), jnp.float32)
        for z in range(zrows // zc):
            pltpu.sync_copy(
                zbuf, shared.at[pl.ds(sid * zrows + z * zc, zc)])
        plsc.subcore_barrier()

        def start_in(ci, p):
            pltpu.async_copy(spkt_hbm.at[pl.ds(rb0 + ci * kr, kr)],
                             val_v.at[p], insems[p])
            pltpu.async_copy(tgt3_hbm.at[pl.ds(rb0 + ci * kr, kr)],
                             idx_v.at[p], insems[p])

        def wait_in(p):
            pltpu.make_async_copy(spkt_hbm.at[pl.ds(0, kr)], val_v.at[p],
                                  insems[p]).wait()
            pltpu.make_async_copy(tgt3_hbm.at[pl.ds(0, kr)], idx_v.at[p],
                                  insems[p]).wait()

        def issue_scatter(p):
            for j in range(kr):
                pltpu.async_copy(val_v.at[p, j],
                                 shared.at[idx_v.at[p, j]], scsems[p],
                                 add=True)

        def drain_scatter(p):
            for j in range(kr):
                pltpu.make_async_copy(val_v.at[p, j],
                                      shared.at[idx_v.at[p, j]],
                                      scsems[p]).wait()

        start_in(0, 0)

        def pair_body(i, _):
            start_in(2 * i + 1, 1)
            wait_in(0)
            issue_scatter(0)
            drain_scatter(0)

            @pl.when(2 * i + 2 < nchunks)
            def _():
                start_in(2 * i + 2, 0)
            wait_in(1)
            issue_scatter(1)
            drain_scatter(1)
            return 0
        lax.fori_loop(0, nchunks // 2, pair_body, 0)

        plsc.subcore_barrier()

        @pl.when(sid == 0)
        def _():
            pltpu.sync_copy(shared, isynt_hbm.at[cid])

    return k


def _neuron_kernel(nh, no, tail_pad, isynt_ref, vm_ref, acc_ref, inp_ref,
                   phase_ref, vexct_ref, tail_ref):
    inject = (phase_ref[...] == 2).astype(jnp.float32)
    inp = inp_ref[...]
    b = inp.shape[0]
    i_syn = (isynt_ref[0] + isynt_ref[1]).T
    i_inj = jnp.concatenate(
        [inp * inject, jnp.zeros((b, no), jnp.float32)], axis=1)
    i_syn = i_syn + i_inj
    vm = vm_ref[...]
    vm1 = vm + (i_syn - vm) * (_DT / _TAU)
    v_exc = jnp.maximum(0.0, vm1 - _THRESH)
    fired = (v_exc > 0).astype(jnp.float32)
    vm2 = vm1 - vm1 * fired + 0.2 * fired
    acc1 = acc_ref[...] + vm1[:, -no:]
    spike_rate = jnp.mean(fired, axis=1, keepdims=True)
    input_norm = jnp.sqrt(jnp.sum(inp * inp, axis=1, keepdims=True)) * inject
    vexct_ref[...] = v_exc.T
    tail_ref[...] = jnp.concatenate(
        [vm2, acc1, inject, spike_rate, input_norm,
         jnp.zeros((b, tail_pad), jnp.float32)], axis=1)


def _make_sc_gather(b, e, n, ce):
    eper = e // _NW
    nchunks = eper // ce
    assert nchunks % 2 == 0
    kr = ce // _IR
    srows = n // _NS

    @functools.partial(
        pl.kernel, mesh=_mesh(),
        out_type=jax.ShapeDtypeStruct((e // _IR, _IR, b), jnp.float32),
        scratch_types=[
            pltpu.VMEM((2, kr, _IR, b), jnp.float32),
            pltpu.VMEM((2, kr, _IR), jnp.int32),
            pltpu.VMEM_SHARED((n, b), jnp.float32),
            pltpu.SemaphoreType.DMA,
            pltpu.SemaphoreType.DMA,
            pltpu.SemaphoreType.DMA,
            pltpu.SemaphoreType.DMA,
            pltpu.SemaphoreType.DMA,
            pltpu.SemaphoreType.DMA,
        ],
        compiler_params=_sc_params(),
    )
    def k(vexct_hbm, src3_hbm, gt_hbm, g_v, idx_v, shared,
          in0, in1, ga0, ga1, out0, out1):
        cid = lax.axis_index("c")
        sid = lax.axis_index("s")
        wid = sid * _NC + cid
        rb0 = (wid * eper) // _IR
        insems = (in0, in1)
        gasems = (ga0, ga1)
        outsems = (out0, out1)

        pltpu.sync_copy(vexct_hbm.at[pl.ds(sid * srows, srows)],
                        shared.at[pl.ds(sid * srows, srows)])
        plsc.subcore_barrier()

        def start_in(ci, p):
            pltpu.async_copy(src3_hbm.at[pl.ds(rb0 + ci * kr, kr)],
                             idx_v.at[p], insems[p])

        def wait_in(p):
            pltpu.make_async_copy(src3_hbm.at[pl.ds(0, kr)], idx_v.at[p],
                                  insems[p]).wait()

        def issue_gather(p):
            for j in range(kr):
                pltpu.async_copy(shared.at[idx_v.at[p, j]],
                                 g_v.at[p, j], gasems[p])

        def drain_gather(p):
            for j in range(kr):
                pltpu.make_async_copy(shared.at[idx_v.at[p, j]],
                                      g_v.at[p, j], gasems[p]).wait()

        def start_out(ci, p):
            pltpu.async_copy(g_v.at[p],
                             gt_hbm.at[pl.ds(rb0 + ci * kr, kr)],
                             outsems[p])

        def drain_out(p):
            pltpu.make_async_copy(g_v.at[p], gt_hbm.at[pl.ds(0, kr)],
                                  outsems[p]).wait()

        start_in(0, 0)

        def pair_body(i, _):
            start_in(2 * i + 1, 1)
            wait_in(0)

            @pl.when(i > 0)
            def _():
                drain_out(0)
            issue_gather(0)
            drain_gather(0)
            start_out(2 * i, 0)

            @pl.when(2 * i + 2 < nchunks)
            def _():
                start_in(2 * i + 2, 0)
            wait_in(1)

            @pl.when(i > 0)
            def _():
                drain_out(1)
            issue_gather(1)
            drain_gather(1)
            start_out(2 * i + 1, 1)
            return 0
        lax.fori_loop(0, nchunks // 2, pair_body, 0)
        drain_out(0)
        drain_out(1)

    return k


def _pass2_kernel(n_st_blocks, st_lo_ref, sv_lo_ref, l_lo_ref, st_hi_ref,
                  sv_hi_ref, l_hi_ref, gt_ref, tail_ref, out_ref,
                  stash_sthi, stash_svlo, stash_svhi):
    i = pl.program_id(0)
    last = 2 * n_st_blocks

    def half(st_ref, sv_ref, l_ref, g):
        st = st_ref[...]
        sv = sv_ref[...]
        lvals = l_ref[0]
        arrived = jnp.abs(st - lvals) <= (_ATOL + _RTOL * jnp.abs(lvals))
        stz = jnp.where(arrived, 0.0, st)
        svz = jnp.where(arrived, 0.0, sv)
        new = (g > 0) & (st == 0)
        st2 = stz + jnp.where(stz > 0, _DT * _VMAX, 0.0) \
                  + jnp.where(new, _DT * _VMAX, 0.0)
        sv2 = svz + jnp.where(new, g, 0.0)
        return st2, sv2

    @pl.when((i < last) & (i % 4 == 0))
    def _():
        b = st_lo_ref.shape[0]
        gt = gt_ref[...]
        st2_lo, sv2_lo = half(st_lo_ref, sv_lo_ref, l_lo_ref, gt[:, :b].T)
        st2_hi, sv2_hi = half(st_hi_ref, sv_hi_ref, l_hi_ref, gt[:, b:].T)
        out_ref[...] = st2_lo
        stash_sthi[...] = st2_hi
        stash_svlo[...] = sv2_lo
        stash_svhi[...] = sv2_hi

    @pl.when((i < last) & (i % 4 == 1))
    def _():
        out_ref[...] = stash_sthi[...]

    @pl.when((i < last) & (i % 4 == 2))
    def _():
        out_ref[...] = stash_svlo[...]

    @pl.when((i < last) & (i % 4 == 3))
    def _():
        out_ref[...] = stash_svhi[...]

    @pl.when(i == last)
    def _():
        out_ref[...] = tail_ref[...]


def _pack_idx(idx, e):
    return idx.reshape(2, e // _IR, _IR // 2).transpose(1, 2, 0).reshape(
        e // _IR, _IR)


def kernel(syn_travel, syn_value, vm, acc, input_current, L_e, W_e, phase,
           src, tgt):
    b, e = syn_travel.shape
    n = vm.shape[1]
    nh = input_current.shape[1]
    no = acc.shape[1]
    f32 = jnp.float32
    half = e // 2

    eba = _EB_A
    nblk_a = half // eba
    l3 = L_e.reshape(2 * nblk_a, 1, eba)
    w3 = W_e.reshape(2 * nblk_a, 1, eba)

    def lo2(i):
        return (0, i)

    def hi2(i):
        return (0, i + nblk_a)

    def lo3(i):
        return (i, 0, 0)

    def hi3(i):
        return (i + nblk_a, 0, 0)

    spikes_t = pl.pallas_call(
        _spikes_kernel,
        grid=(nblk_a,),
        in_specs=[
            pl.BlockSpec((b, eba), lo2),
            pl.BlockSpec((b, eba), lo2),
            pl.BlockSpec((1, 1, eba), lo3),
            pl.BlockSpec((1, 1, eba), lo3),
            pl.BlockSpec((b, eba), hi2),
            pl.BlockSpec((b, eba), hi2),
            pl.BlockSpec((1, 1, eba), hi3),
            pl.BlockSpec((1, 1, eba), hi3),
        ],
        out_specs=pl.BlockSpec((eba, _PB), lambda i: (i, 0)),
        out_shape=jax.ShapeDtypeStruct((half, _PB), f32),
    )(syn_travel, syn_value, l3, w3, syn_travel, syn_value, l3, w3)

    ce = 384
    isynt = _make_sc_scatter(b, e, n, ce)(
        spikes_t.reshape(e // _IR, _IR, b), _pack_idx(tgt, e))

    tail_cols = n + no + 3
    tail_pad = _EB_C - tail_cols
    vexct, tail = pl.pallas_call(
        functools.partial(_neuron_kernel, nh, no, tail_pad),
        out_shape=[
            jax.ShapeDtypeStruct((n, b), f32),
            jax.ShapeDtypeStruct((b, _EB_C), f32),
        ],
    )(isynt, vm, acc, input_current, phase.reshape(b, 1))

    gathered_t = _make_sc_gather(b, e, n, ce)(
        vexct, _pack_idx(src, e)).reshape(half, _PB)

    ebc = _EB_C
    nblk_c = e // ebc
    nhalf_c = nblk_c // 2
    out_cols = 2 * e + tail_cols
    l3c = L_e.reshape(nblk_c, 1, ebc)

    def q_of(i):
        return jnp.minimum(i // 4, nhalf_c - 1)

    def lo_map2(i):
        return (0, q_of(i))

    def hi_map2(i):
        return (0, nhalf_c + q_of(i))

    def lo_map3(i):
        return (q_of(i), 0, 0)

    def hi_map3(i):
        return (nhalf_c + q_of(i), 0, 0)

    def gt_map(i):
        return (q_of(i), 0)

    def out_map(i):
        return (0, jnp.where(i == 2 * nblk_c, 2 * nblk_c,
                             (i % 4) * nhalf_c + i // 4))

    out = pl.pallas_call(
        functools.partial(_pass2_kernel, nblk_c),
        grid=(2 * nblk_c + 1,),
        in_specs=[
            pl.BlockSpec((b, ebc), lo_map2),
            pl.BlockSpec((b, ebc), lo_map2),
            pl.BlockSpec((1, 1, ebc), lo_map3),
            pl.BlockSpec((b, ebc), hi_map2),
            pl.BlockSpec((b, ebc), hi_map2),
            pl.BlockSpec((1, 1, ebc), hi_map3),
            pl.BlockSpec((ebc, _PB), gt_map),
            pl.BlockSpec((b, _EB_C), lambda i: (0, 0)),
        ],
        out_specs=pl.BlockSpec((b, ebc), out_map),
        out_shape=jax.ShapeDtypeStruct((b, out_cols), f32),
        scratch_shapes=[pltpu.VMEM((b, ebc), f32),
                        pltpu.VMEM((b, ebc), f32),
                        pltpu.VMEM((b, ebc), f32)],
    )(syn_travel, syn_value, l3c, syn_travel, syn_value, l3c,
      gathered_t, tail)

    return out
